# Initial kernel scaffold; baseline (speedup 1.0000x reference)
import jax
import jax.numpy as jnp
from jax.experimental import pallas as pl


def kernel(input_ids, token_type_ids, word_table, type_table, gamma, beta):
    raise NotImplementedError("write your pallas kernel here")


# speedup vs baseline: 1.4772x; 1.4772x over previous
import functools

import jax
import jax.numpy as jnp
from jax import lax
from jax.experimental import pallas as pl
from jax.experimental.pallas import tpu as pltpu
from jax.experimental.pallas import tpu_sc as plsc

_EPS = 1e-12


@functools.lru_cache(maxsize=None)
def _make_sc_gather(V, D, B):
    info = plsc.get_sparse_core_info()
    NC, NS = info.num_cores, info.num_subcores
    NW = NC * NS
    assert B % NW == 0
    b_per_w = B // NW
    C = 32
    assert b_per_w % C == 0
    NCH = b_per_w // C
    mesh = plsc.VectorSubcoreMesh(core_axis_name="c", subcore_axis_name="s")

    @functools.partial(
        pl.kernel,
        mesh=mesh,
        out_type=jax.ShapeDtypeStruct((B, D), jnp.float32),
        scratch_types=[
            pltpu.VMEM((NCH, C), jnp.int32),
            pltpu.VMEM((C, D), jnp.float32),
            pltpu.VMEM((C, D), jnp.float32),
            pltpu.SemaphoreType.DMA,
            pltpu.SemaphoreType.DMA,
            pltpu.SemaphoreType.DMA,
            pltpu.SemaphoreType.DMA,
        ],
    )
    def gather_k(table_hbm, idx_hbm, out_hbm, idx_v, buf_a, buf_b,
                 gs_a, gs_b, os_a, os_b):
        wid = lax.axis_index("s") * NC + lax.axis_index("c")
        base = wid * b_per_w
        for c in range(NCH):
            pltpu.sync_copy(idx_hbm.at[pl.ds(base + c * C, C)], idx_v.at[c])
        bufs = (buf_a, buf_b)
        gs = (gs_a, gs_b)
        os = (os_a, os_b)
        g = [None] * NCH
        o = [None] * NCH
        g[0] = pltpu.async_copy(table_hbm.at[idx_v.at[0]], bufs[0], gs[0])
        for c in range(NCH):
            nxt = c + 1
            if nxt < NCH:
                if nxt >= 2:
                    o[nxt - 2].wait()
                g[nxt] = pltpu.async_copy(
                    table_hbm.at[idx_v.at[nxt]], bufs[nxt % 2], gs[nxt % 2])
            g[c].wait()
            o[c] = pltpu.async_copy(
                bufs[c % 2], out_hbm.at[pl.ds(base + c * C, C)], os[c % 2])
        o[NCH - 2].wait()
        o[NCH - 1].wait()

    return gather_k


def _ln_body(x_ref, tt_ref, tp_ref, g_ref, b_ref, o_ref):
    x = x_ref[...]
    t0 = tp_ref[0:1, :]
    t1 = tp_ref[1:2, :]
    f = tt_ref[...]
    x = x + t0 + f * (t1 - t0)
    mean = jnp.mean(x, axis=-1, keepdims=True)
    xc = x - mean
    var = jnp.mean(xc * xc, axis=-1, keepdims=True)
    o_ref[...] = xc * lax.rsqrt(var + _EPS) * g_ref[...] + b_ref[...]


@functools.lru_cache(maxsize=None)
def _make_tc_ln(B, D, BR=256):
    assert B % BR == 0
    grid = (B // BR,)
    return pl.pallas_call(
        _ln_body,
        grid=grid,
        in_specs=[
            pl.BlockSpec((BR, D), lambda i: (i, 0)),
            pl.BlockSpec((BR, 1), lambda i: (i, 0)),
            pl.BlockSpec((2, D), lambda i: (0, 0)),
            pl.BlockSpec((1, D), lambda i: (0, 0)),
            pl.BlockSpec((1, D), lambda i: (0, 0)),
        ],
        out_specs=pl.BlockSpec((BR, D), lambda i: (i, 0)),
        out_shape=jax.ShapeDtypeStruct((B, D), jnp.float32),
    )


def kernel(input_ids, token_type_ids, word_table, type_table, gamma, beta):
    Bt = input_ids.shape[0] * input_ids.shape[1]
    V, D = word_table.shape
    ids = input_ids.reshape(-1).astype(jnp.int32)
    gathered = _make_sc_gather(V, D, Bt)(word_table, ids)
    ttf = token_type_ids.reshape(-1, 1).astype(jnp.float32)
    out = _make_tc_ln(Bt, D)(
        gathered, ttf, type_table,
        gamma.reshape(1, -1), beta.reshape(1, -1))
    return out.reshape(input_ids.shape + (D,))


# speedup vs baseline: 1.5595x; 1.0557x over previous
import functools

import jax
import jax.numpy as jnp
from jax import lax
from jax.experimental import pallas as pl
from jax.experimental.pallas import tpu as pltpu
from jax.experimental.pallas import tpu_sc as plsc

_EPS = 1e-12
_BR = 256


@functools.lru_cache(maxsize=None)
def _make_sc_gather(V, D, B):
    info = plsc.get_sparse_core_info()
    NC, NS = info.num_cores, info.num_subcores
    NW = NC * NS
    assert B % NW == 0
    b_per_w = B // NW
    C = 32
    assert b_per_w % C == 0
    NCH = b_per_w // C
    assert NCH >= 2
    mesh = plsc.VectorSubcoreMesh(core_axis_name="c", subcore_axis_name="s")

    @functools.partial(
        pl.kernel,
        mesh=mesh,
        out_type=jax.ShapeDtypeStruct((B, D), jnp.float32),
        scratch_types=[
            pltpu.VMEM((NCH, C), jnp.int32),
            pltpu.VMEM((C, D), jnp.float32),
            pltpu.VMEM((C, D), jnp.float32),
            pltpu.SemaphoreType.DMA,
            pltpu.SemaphoreType.DMA,
            pltpu.SemaphoreType.DMA,
            pltpu.SemaphoreType.DMA,
        ],
    )
    def gather_k(table_hbm, idx_hbm, out_hbm, idx_v, buf_a, buf_b,
                 gs_a, gs_b, os_a, os_b):
        wid = lax.axis_index("s") * NC + lax.axis_index("c")
        base = wid * b_per_w
        for c in range(NCH):
            pltpu.sync_copy(idx_hbm.at[pl.ds(base + c * C, C)], idx_v.at[c])
        bufs = (buf_a, buf_b)
        gs = (gs_a, gs_b)
        os = (os_a, os_b)
        g = [None] * NCH
        o = [None] * NCH
        g[0] = pltpu.async_copy(table_hbm.at[idx_v.at[0]], bufs[0], gs[0])
        for c in range(NCH):
            nxt = c + 1
            if nxt < NCH:
                if nxt >= 2:
                    o[nxt - 2].wait()
                g[nxt] = pltpu.async_copy(
                    table_hbm.at[idx_v.at[nxt]], bufs[nxt % 2], gs[nxt % 2])
            g[c].wait()
            o[c] = pltpu.async_copy(
                bufs[c % 2], out_hbm.at[pl.ds(base + c * C, C)], os[c % 2])
        o[NCH - 2].wait()
        o[NCH - 1].wait()

    return gather_k


def _ln_core(x_ref, tt_ref, tp_ref, g_ref, b_ref, o_ref):
    x = x_ref[...]
    t0 = tp_ref[0:1, :]
    t1 = tp_ref[1:2, :]
    f = tt_ref[...]
    x = x + t0 + f * (t1 - t0)
    mean = jnp.mean(x, axis=-1, keepdims=True)
    xc = x - mean
    var = jnp.mean(xc * xc, axis=-1, keepdims=True)
    o_ref[...] = xc * lax.rsqrt(var + _EPS) * g_ref[...] + b_ref[...]


def _ln_body_alias(buf_ref, x_ref, tt_ref, tp_ref, g_ref, b_ref, o_ref):
    del buf_ref
    _ln_core(x_ref, tt_ref, tp_ref, g_ref, b_ref, o_ref)


@functools.lru_cache(maxsize=None)
def _make_tc_ln(B_total, B_chunk, D, block_off, aliased):
    assert B_chunk % _BR == 0
    grid = (B_chunk // _BR,)
    data_specs = [
        pl.BlockSpec((_BR, D), lambda i: (i, 0)),
        pl.BlockSpec((_BR, 1), lambda i: (i, 0)),
        pl.BlockSpec((2, D), lambda i: (0, 0)),
        pl.BlockSpec((1, D), lambda i: (0, 0)),
        pl.BlockSpec((1, D), lambda i: (0, 0)),
    ]
    if aliased:
        in_specs = [pl.BlockSpec(memory_space=pl.ANY)] + data_specs
        body = _ln_body_alias
        aliases = {0: 0}
    else:
        in_specs = data_specs
        body = _ln_core
        aliases = {}
    return pl.pallas_call(
        body,
        grid=grid,
        in_specs=in_specs,
        out_specs=pl.BlockSpec((_BR, D), lambda i: (block_off + i, 0)),
        out_shape=jax.ShapeDtypeStruct((B_total, D), jnp.float32),
        input_output_aliases=aliases,
    )


def kernel(input_ids, token_type_ids, word_table, type_table, gamma, beta):
    NB, S = input_ids.shape
    V, D = word_table.shape
    Bt = NB * S
    gamma2 = gamma.reshape(1, -1)
    beta2 = beta.reshape(1, -1)
    sc_gather = _make_sc_gather(V, D, S)
    blocks_per_chunk = S // _BR

    gathered = []
    for b in range(NB):
        ids_b = input_ids[b].astype(jnp.int32)
        gathered.append(sc_gather(word_table, ids_b))

    out = None
    for b in range(NB):
        ttf = token_type_ids[b].reshape(-1, 1).astype(jnp.float32)
        ln = _make_tc_ln(Bt, S, D, b * blocks_per_chunk, b > 0)
        if b == 0:
            out = ln(gathered[b], ttf, type_table, gamma2, beta2)
        else:
            out = ln(out, gathered[b], ttf, type_table, gamma2, beta2)
    return out.reshape(NB, S, D)
